# baseline (speedup 1.0000x reference)
import functools

import jax
import jax.numpy as jnp
from jax import lax
from jax.experimental import pallas as pl
from jax.experimental.pallas import tpu as pltpu
from jax.experimental.pallas import tpu_sc as plsc

TAU = 1.0
B = 2048
E = 8
D_COND = 1024
D_NOISE = 128
D_IN = D_NOISE + D_COND
H = 1024
D_OUT = 56 * 30
BLOCK = 128
NBLK = 23
NSLOT = NBLK * BLOCK
BE_PAD = 128

NW = 32
TPW = B // NW


def _router_body(cond_ref, gum_ref, wr_ref, br_ref,
                 logits_ref, gates_ref, counts_ref, dest_ref, be_ref):
    eps = 1e-10
    logits = jnp.dot(cond_ref[...], wr_ref[...],
                     preferred_element_type=jnp.float32) + br_ref[...]
    logits_ref[...] = logits
    g = -jnp.log(-jnp.log(gum_ref[...] + eps) + eps)
    z = (logits + g) / TAU
    m = jnp.max(z, axis=1, keepdims=True)
    ez = jnp.exp(z - m)
    gates_ref[...] = ez / jnp.sum(ez, axis=1, keepdims=True)

    is_max = (z == m).astype(jnp.float32)
    ir = lax.broadcasted_iota(jnp.int32, (E, E), 0)
    ic = lax.broadcasted_iota(jnp.int32, (E, E), 1)
    u_incl = (ir <= ic).astype(jnp.float32)
    u_strict = (ir < ic).astype(jnp.float32)
    prefix = jnp.dot(is_max, u_incl, preferred_element_type=jnp.float32)
    onehot = is_max * (prefix == 1.0).astype(jnp.float32)

    counts = jnp.sum(onehot, axis=0, keepdims=True)
    counts_ref[...] = counts / B

    c = onehot
    k = 1
    while k < B:
        sh = jnp.concatenate([jnp.zeros((k, E), jnp.float32), c[: B - k]],
                             axis=0)
        c = c + sh
        k *= 2
    excl = c - onehot
    rank = jnp.sum(excl * onehot, axis=1, keepdims=True)

    nb = jnp.floor((counts + (BLOCK - 1)) * (1.0 / BLOCK))
    starts = jnp.dot(nb, u_strict, preferred_element_type=jnp.float32)
    ends = jnp.dot(nb, u_incl, preferred_element_type=jnp.float32)
    base = jnp.sum(onehot * starts, axis=1, keepdims=True) * BLOCK
    dest_ref[...] = (base + rank).astype(jnp.int32)

    biota = lax.broadcasted_iota(jnp.int32, (BE_PAD, E), 0).astype(jnp.float32)
    ge = (biota >= ends).astype(jnp.float32)
    be = jnp.minimum(jnp.sum(ge, axis=1, keepdims=True), E - 1)
    be_ref[...] = be.astype(jnp.int32)


def _router(cond, gumbel_u, W_router, b_router):
    return pl.pallas_call(
        _router_body,
        out_shape=(
            jax.ShapeDtypeStruct((B, E), jnp.float32),
            jax.ShapeDtypeStruct((B, E), jnp.float32),
            jax.ShapeDtypeStruct((1, E), jnp.float32),
            jax.ShapeDtypeStruct((B, 1), jnp.int32),
            jax.ShapeDtypeStruct((BE_PAD, 1), jnp.int32),
        ),
    )(cond, gumbel_u, W_router, b_router.reshape(1, E))


def _dispatch_body(noise_hbm, cond_hbm, dest_hbm, xs_hbm, idx_v, rows_v, sem):
    wid = lax.axis_index("s") * 2 + lax.axis_index("c")
    base = wid * TPW
    pltpu.sync_copy(dest_hbm.at[pl.ds(base, TPW)], idx_v)
    pltpu.sync_copy(noise_hbm.at[pl.ds(base, TPW)],
                    rows_v.at[:, pl.ds(0, D_NOISE)])
    pltpu.sync_copy(cond_hbm.at[pl.ds(base, TPW)],
                    rows_v.at[:, pl.ds(D_NOISE, D_COND)])
    pltpu.async_copy(rows_v, xs_hbm.at[idx_v], sem).wait()


def _dispatch(noise, cond, dest):
    mesh = plsc.VectorSubcoreMesh(core_axis_name="c", subcore_axis_name="s",
                                   num_cores=2, num_subcores=16)
    return pl.kernel(
        _dispatch_body,
        out_type=jax.ShapeDtypeStruct((NSLOT, D_IN), jnp.float32),
        mesh=mesh,
        scratch_types=[
            pltpu.VMEM((TPW,), jnp.int32),
            pltpu.VMEM((TPW, D_IN), jnp.float32),
            pltpu.SemaphoreType.DMA,
        ],
    )(noise, cond, dest)


def _combine_body(ys_hbm, dest_hbm, out_hbm, idx_v, rows_v, sem):
    wid = lax.axis_index("s") * 2 + lax.axis_index("c")
    base = wid * TPW
    pltpu.sync_copy(dest_hbm.at[pl.ds(base, TPW)], idx_v)
    pltpu.async_copy(ys_hbm.at[idx_v], rows_v, sem).wait()
    pltpu.sync_copy(rows_v, out_hbm.at[pl.ds(base, TPW)])


def _combine(ys, dest):
    mesh = plsc.VectorSubcoreMesh(core_axis_name="c", subcore_axis_name="s",
                                   num_cores=2, num_subcores=16)
    return pl.kernel(
        _combine_body,
        out_type=jax.ShapeDtypeStruct((B, D_OUT), jnp.float32),
        mesh=mesh,
        compiler_params=pltpu.CompilerParams(use_tc_tiling_on_sc=False),
        scratch_types=[
            pltpu.VMEM((TPW,), jnp.int32),
            pltpu.VMEM((TPW, D_OUT), jnp.float32),
            pltpu.SemaphoreType.DMA,
        ],
    )(ys, dest)


def _experts_body(be_ref, xs_ref, w1_ref, b1_ref, w2_ref, b2_ref, out_ref):
    h = jnp.dot(xs_ref[...].astype(jnp.bfloat16), w1_ref[0],
                preferred_element_type=jnp.float32) + b1_ref[0]
    h = jnp.maximum(h, 0.0)
    y = jnp.dot(h.astype(jnp.bfloat16), w2_ref[0],
                preferred_element_type=jnp.float32) + b2_ref[0]
    out_ref[...] = jnp.tanh(y)


def _experts(be, xs, W1, b1, W2, b2):
    grid_spec = pltpu.PrefetchScalarGridSpec(
        num_scalar_prefetch=1,
        grid=(NBLK,),
        in_specs=[
            pl.BlockSpec((BLOCK, D_IN), lambda i, be: (i, 0)),
            pl.BlockSpec((1, D_IN, H), lambda i, be: (be[i, 0], 0, 0)),
            pl.BlockSpec((1, 1, H), lambda i, be: (be[i, 0], 0, 0)),
            pl.BlockSpec((1, H, D_OUT), lambda i, be: (be[i, 0], 0, 0)),
            pl.BlockSpec((1, 1, D_OUT), lambda i, be: (be[i, 0], 0, 0)),
        ],
        out_specs=pl.BlockSpec((BLOCK, D_OUT), lambda i, be: (i, 0)),
    )
    return pl.pallas_call(
        _experts_body,
        grid_spec=grid_spec,
        out_shape=jax.ShapeDtypeStruct((NSLOT, D_OUT), jnp.float32),
    )(be, xs, W1.astype(jnp.bfloat16), b1.reshape(E, 1, H),
      W2.astype(jnp.bfloat16), b2.reshape(E, 1, D_OUT))


def kernel(cond, noise, gumbel_u, W_router, b_router, W1, b1, W2, b2):
    logits, gates_soft, counts_adj, dest2, be = _router(
        cond, gumbel_u, W_router, b_router)
    dest = dest2.reshape(B)
    xs = _dispatch(noise, cond, dest)
    ys = _experts(be, xs, W1, b1, W2, b2)
    fake_images = _combine(ys, dest)
    return fake_images, gates_soft, logits, counts_adj.reshape(E)


# speedup vs baseline: 1.0850x; 1.0850x over previous
import functools

import jax
import jax.numpy as jnp
from jax import lax
from jax.experimental import pallas as pl
from jax.experimental.pallas import tpu as pltpu
from jax.experimental.pallas import tpu_sc as plsc

TAU = 1.0
B = 2048
E = 8
D_COND = 1024
D_NOISE = 128
D_IN = D_NOISE + D_COND
H = 1024
D_OUT = 56 * 30
BLOCK = 128
NBLK = 23
NSLOT = NBLK * BLOCK
BE_PAD = 128

NW = 32
TPW = B // NW


def _router_body(cond_ref, gum_ref, wr_ref, br_ref,
                 logits_ref, gates_ref, counts_ref, dest_ref, be_ref):
    eps = 1e-10
    logits = jnp.dot(cond_ref[...], wr_ref[...],
                     preferred_element_type=jnp.float32) + br_ref[...]
    logits_ref[...] = logits
    g = -jnp.log(-jnp.log(gum_ref[...] + eps) + eps)
    z = (logits + g) / TAU
    m = jnp.max(z, axis=1, keepdims=True)
    ez = jnp.exp(z - m)
    gates_ref[...] = ez / jnp.sum(ez, axis=1, keepdims=True)

    is_max = (z == m).astype(jnp.float32)
    ir = lax.broadcasted_iota(jnp.int32, (E, E), 0)
    ic = lax.broadcasted_iota(jnp.int32, (E, E), 1)
    u_incl = (ir <= ic).astype(jnp.float32)
    u_strict = (ir < ic).astype(jnp.float32)
    prefix = jnp.dot(is_max, u_incl, preferred_element_type=jnp.float32)
    onehot = is_max * (prefix == 1.0).astype(jnp.float32)

    counts = jnp.sum(onehot, axis=0, keepdims=True)
    counts_ref[...] = counts / B

    c = onehot
    k = 1
    while k < B:
        sh = jnp.concatenate([jnp.zeros((k, E), jnp.float32), c[: B - k]],
                             axis=0)
        c = c + sh
        k *= 2
    excl = c - onehot
    rank = jnp.sum(excl * onehot, axis=1, keepdims=True)

    nb = jnp.floor((counts + (BLOCK - 1)) * (1.0 / BLOCK))
    starts = jnp.dot(nb, u_strict, preferred_element_type=jnp.float32)
    ends = jnp.dot(nb, u_incl, preferred_element_type=jnp.float32)
    base = jnp.sum(onehot * starts, axis=1, keepdims=True) * BLOCK
    dest_ref[...] = (base + rank).astype(jnp.int32)

    biota = lax.broadcasted_iota(jnp.int32, (BE_PAD, E), 0).astype(jnp.float32)
    ge = (biota >= ends).astype(jnp.float32)
    be = jnp.minimum(jnp.sum(ge, axis=1, keepdims=True), E - 1)
    be_ref[...] = be.astype(jnp.int32)


def _router(cond, gumbel_u, W_router, b_router):
    return pl.pallas_call(
        _router_body,
        out_shape=(
            jax.ShapeDtypeStruct((B, E), jnp.float32),
            jax.ShapeDtypeStruct((B, E), jnp.float32),
            jax.ShapeDtypeStruct((1, E), jnp.float32),
            jax.ShapeDtypeStruct((B, 1), jnp.int32),
            jax.ShapeDtypeStruct((BE_PAD, 1), jnp.int32),
        ),
    )(cond, gumbel_u, W_router, b_router.reshape(1, E))


def _dispatch_body(noise_hbm, cond_hbm, dest_hbm, xs_hbm, idx_v, rows_v, sem):
    wid = lax.axis_index("s") * 2 + lax.axis_index("c")
    base = wid * TPW
    pltpu.sync_copy(dest_hbm.at[pl.ds(base, TPW)], idx_v)
    pltpu.sync_copy(noise_hbm.at[pl.ds(base, TPW)],
                    rows_v.at[:, pl.ds(0, D_NOISE)])
    pltpu.sync_copy(cond_hbm.at[pl.ds(base, TPW)],
                    rows_v.at[:, pl.ds(D_NOISE, D_COND)])
    pltpu.async_copy(rows_v, xs_hbm.at[idx_v], sem).wait()


def _dispatch(noise, cond, dest):
    mesh = plsc.VectorSubcoreMesh(core_axis_name="c", subcore_axis_name="s",
                                   num_cores=2, num_subcores=16)
    return pl.kernel(
        _dispatch_body,
        out_type=jax.ShapeDtypeStruct((NSLOT, D_IN), jnp.float32),
        mesh=mesh,
        scratch_types=[
            pltpu.VMEM((TPW,), jnp.int32),
            pltpu.VMEM((TPW, D_IN), jnp.float32),
            pltpu.SemaphoreType.DMA,
        ],
    )(noise, cond, dest)


def _combine_body(ys_hbm, dest_hbm, out_hbm, idx_v, rows_v, sem):
    wid = lax.axis_index("s") * 2 + lax.axis_index("c")
    base = wid * TPW
    pltpu.sync_copy(dest_hbm.at[pl.ds(base, TPW)], idx_v)
    pltpu.async_copy(ys_hbm.at[idx_v], rows_v, sem).wait()
    pltpu.sync_copy(rows_v, out_hbm.at[pl.ds(base, TPW)])


def _combine(ys, dest):
    mesh = plsc.VectorSubcoreMesh(core_axis_name="c", subcore_axis_name="s",
                                   num_cores=2, num_subcores=16)
    return pl.kernel(
        _combine_body,
        out_type=jax.ShapeDtypeStruct((B, D_OUT), jnp.float32),
        mesh=mesh,
        compiler_params=pltpu.CompilerParams(use_tc_tiling_on_sc=False),
        scratch_types=[
            pltpu.VMEM((TPW,), jnp.int32),
            pltpu.VMEM((TPW, D_OUT), jnp.float32),
            pltpu.SemaphoreType.DMA,
        ],
    )(ys, dest)


def _experts_body(be_ref, xs_ref, w1_ref, b1_ref, w2_ref, b2_ref, out_ref):
    h = jnp.dot(xs_ref[...], w1_ref[0],
                preferred_element_type=jnp.float32) + b1_ref[0]
    h = jnp.maximum(h, 0.0)
    y = jnp.dot(h, w2_ref[0],
                preferred_element_type=jnp.float32) + b2_ref[0]
    out_ref[...] = jnp.tanh(y)


def _experts(be, xs, W1, b1, W2, b2):
    grid_spec = pltpu.PrefetchScalarGridSpec(
        num_scalar_prefetch=1,
        grid=(NBLK,),
        in_specs=[
            pl.BlockSpec((BLOCK, D_IN), lambda i, be: (i, 0)),
            pl.BlockSpec((1, D_IN, H), lambda i, be: (be[i, 0], 0, 0)),
            pl.BlockSpec((1, 1, H), lambda i, be: (be[i, 0], 0, 0)),
            pl.BlockSpec((1, H, D_OUT), lambda i, be: (be[i, 0], 0, 0)),
            pl.BlockSpec((1, 1, D_OUT), lambda i, be: (be[i, 0], 0, 0)),
        ],
        out_specs=pl.BlockSpec((BLOCK, D_OUT), lambda i, be: (i, 0)),
    )
    return pl.pallas_call(
        _experts_body,
        grid_spec=grid_spec,
        out_shape=jax.ShapeDtypeStruct((NSLOT, D_OUT), jnp.float32),
    )(be, xs, W1, b1.reshape(E, 1, H), W2, b2.reshape(E, 1, D_OUT))


def kernel(cond, noise, gumbel_u, W_router, b_router, W1, b1, W2, b2):
    logits, gates_soft, counts_adj, dest2, be = _router(
        cond, gumbel_u, W_router, b_router)
    dest = dest2.reshape(B)
    xs = _dispatch(noise, cond, dest)
    ys = _experts(be, xs, W1, b1, W2, b2)
    fake_images = _combine(ys, dest)
    return fake_images, gates_soft, logits, counts_adj.reshape(E)


# speedup vs baseline: 1.8213x; 1.6787x over previous
import jax
import jax.numpy as jnp
from jax import lax
from jax.experimental import pallas as pl
from jax.experimental.pallas import tpu as pltpu
from jax.experimental.pallas import tpu_sc as plsc

TAU = 1.0
B = 2048
E = 8
D_COND = 1024
D_NOISE = 128
D_IN = D_NOISE + D_COND
H = 1024
D_OUT = 56 * 30
D_PAD = 1792
BLOCK = 128
NBLK = 23
NSLOT = NBLK * BLOCK
BE_PAD = 128

NW = 32
TPW = B // NW


def _router_body(cond_ref, gum_ref, wr_ref, br_ref,
                 logits_ref, gates_ref, counts_ref, dest_ref, be_ref):
    eps = 1e-10
    logits = lax.dot_general(wr_ref[...], cond_ref[...],
                             (((1,), (1,)), ((), ())),
                             preferred_element_type=jnp.float32) + br_ref[...]
    logits_ref[...] = logits
    g = -jnp.log(-jnp.log(gum_ref[...] + eps) + eps)
    z = (logits + g) / TAU
    m = jnp.max(z, axis=0, keepdims=True)
    ez = jnp.exp(z - m)
    gates_ref[...] = ez / jnp.sum(ez, axis=0, keepdims=True)

    is_max = (z == m).astype(jnp.float32)
    ir = lax.broadcasted_iota(jnp.int32, (E, E), 0)
    ic = lax.broadcasted_iota(jnp.int32, (E, E), 1)
    l_incl = (ic <= ir).astype(jnp.float32)
    l_strict = (ic < ir).astype(jnp.float32)
    prefix = jnp.dot(l_incl, is_max, preferred_element_type=jnp.float32)
    onehot = is_max * (prefix == 1.0).astype(jnp.float32)

    counts = jnp.sum(onehot, axis=1, keepdims=True)
    counts_ref[...] = counts / B

    c = onehot
    k = 1
    while k < B:
        sh = jnp.concatenate([jnp.zeros((E, k), jnp.float32), c[:, : B - k]],
                             axis=1)
        c = c + sh
        k *= 2
    excl = c - onehot
    rank = jnp.sum(excl * onehot, axis=0, keepdims=True)

    nb = jnp.floor((counts + (BLOCK - 1)) * (1.0 / BLOCK))
    starts = jnp.dot(l_strict, nb, preferred_element_type=jnp.float32)
    ends = jnp.dot(l_incl, nb, preferred_element_type=jnp.float32)
    base = jnp.sum(onehot * starts, axis=0, keepdims=True) * BLOCK
    dest_ref[...] = (base + rank).astype(jnp.int32)

    biota = lax.broadcasted_iota(jnp.int32, (E, BE_PAD), 1).astype(jnp.float32)
    ge = (biota >= ends).astype(jnp.float32)
    be = jnp.minimum(jnp.sum(ge, axis=0, keepdims=True), E - 1)
    be_ref[...] = be.astype(jnp.int32)


def _router(cond, gum_t, wr_t, br_c):
    return pl.pallas_call(
        _router_body,
        out_shape=(
            jax.ShapeDtypeStruct((E, B), jnp.float32),
            jax.ShapeDtypeStruct((E, B), jnp.float32),
            jax.ShapeDtypeStruct((E, 1), jnp.float32),
            jax.ShapeDtypeStruct((1, B), jnp.int32),
            jax.ShapeDtypeStruct((1, BE_PAD), jnp.int32),
        ),
    )(cond, gum_t, wr_t, br_c)


def _dispatch_body(noise_hbm, cond_hbm, dest_hbm, xs_hbm, idx_v, rows_v, sem):
    wid = lax.axis_index("s") * 2 + lax.axis_index("c")
    base = wid * TPW
    pltpu.sync_copy(dest_hbm.at[pl.ds(base, TPW)], idx_v)
    pltpu.sync_copy(noise_hbm.at[pl.ds(base, TPW)],
                    rows_v.at[:, pl.ds(0, D_NOISE)])
    pltpu.sync_copy(cond_hbm.at[pl.ds(base, TPW)],
                    rows_v.at[:, pl.ds(D_NOISE, D_COND)])
    pltpu.async_copy(rows_v, xs_hbm.at[idx_v], sem).wait()


def _dispatch(noise, cond, dest):
    mesh = plsc.VectorSubcoreMesh(core_axis_name="c", subcore_axis_name="s",
                                  num_cores=2, num_subcores=16)
    return pl.kernel(
        _dispatch_body,
        out_type=jax.ShapeDtypeStruct((NSLOT, D_IN), jnp.float32),
        mesh=mesh,
        scratch_types=[
            pltpu.VMEM((TPW,), jnp.int32),
            pltpu.VMEM((TPW, D_IN), jnp.float32),
            pltpu.SemaphoreType.DMA,
        ],
    )(noise, cond, dest)


def _combine_body(ys_hbm, dest_hbm, out_hbm, idx_v, rows_v, sem):
    wid = lax.axis_index("s") * 2 + lax.axis_index("c")
    base = wid * TPW
    pltpu.sync_copy(dest_hbm.at[pl.ds(base, TPW)], idx_v)
    pltpu.async_copy(ys_hbm.at[idx_v], rows_v, sem).wait()
    pltpu.sync_copy(rows_v, out_hbm.at[pl.ds(base, TPW)])


def _combine(ys, dest):
    mesh = plsc.VectorSubcoreMesh(core_axis_name="c", subcore_axis_name="s",
                                  num_cores=2, num_subcores=16)
    return pl.kernel(
        _combine_body,
        out_type=jax.ShapeDtypeStruct((B, D_PAD), jnp.float32),
        mesh=mesh,
        scratch_types=[
            pltpu.VMEM((TPW,), jnp.int32),
            pltpu.VMEM((TPW, D_PAD), jnp.float32),
            pltpu.SemaphoreType.DMA,
        ],
    )(ys, dest)


def _experts_body(be_ref, xs_ref, w1_ref, b1_ref, w2t_ref, b2_ref, out_ref):
    h = jnp.dot(xs_ref[...], w1_ref[0],
                preferred_element_type=jnp.float32) + b1_ref[0]
    h = jnp.maximum(h, 0.0)
    y = lax.dot_general(h, w2t_ref[0], (((1,), (1,)), ((), ())),
                        preferred_element_type=jnp.float32) + b2_ref[0]
    out_ref[:, :D_OUT] = jnp.tanh(y)


def _experts(be, xs, W1, b1r, W2t, b2r):
    grid_spec = pltpu.PrefetchScalarGridSpec(
        num_scalar_prefetch=1,
        grid=(NBLK,),
        in_specs=[
            pl.BlockSpec((BLOCK, D_IN), lambda i, be: (i, 0)),
            pl.BlockSpec((1, D_IN, H), lambda i, be: (be[0, i], 0, 0)),
            pl.BlockSpec((1, 1, H), lambda i, be: (be[0, i], 0, 0)),
            pl.BlockSpec((1, D_OUT, H), lambda i, be: (be[0, i], 0, 0)),
            pl.BlockSpec((1, 1, D_OUT), lambda i, be: (be[0, i], 0, 0)),
        ],
        out_specs=pl.BlockSpec((BLOCK, D_PAD), lambda i, be: (i, 0)),
    )
    return pl.pallas_call(
        _experts_body,
        grid_spec=grid_spec,
        out_shape=jax.ShapeDtypeStruct((NSLOT, D_PAD), jnp.float32),
    )(be, xs, W1, b1r, W2t, b2r)


def kernel(cond, noise, gumbel_u, W_router, b_router, W1, b1, W2, b2):
    logits_t, gates_t, counts_c, dest_t, be = _router(
        cond, gumbel_u.T, W_router.T, b_router.reshape(E, 1))
    dest = dest_t.reshape(B)
    xs = _dispatch(noise, cond, dest)
    ys = _experts(be, xs, W1, b1.reshape(E, 1, H),
                  jnp.swapaxes(W2, 1, 2), b2.reshape(E, 1, D_OUT))
    outp = _combine(ys, dest)
    return (outp[:, :D_OUT], gates_t.T, logits_t.T, counts_c.reshape(E))


# speedup vs baseline: 1.8969x; 1.0415x over previous
import jax
import jax.numpy as jnp
from jax import lax
from jax.experimental import pallas as pl
from jax.experimental.pallas import tpu as pltpu
from jax.experimental.pallas import tpu_sc as plsc

TAU = 1.0
B = 2048
E = 8
D_COND = 1024
D_NOISE = 128
D_IN = D_NOISE + D_COND
H = 1024
D_OUT = 56 * 30
D_PAD = 1792
BLOCK = 128
NBLK = 23
NSLOT = NBLK * BLOCK
BE_PAD = 128

NW = 32
TPW = B // NW


def _router_body(cond_ref, gum_ref, wr_ref, br_ref,
                 logits_ref, gates_ref, counts_ref, dest_ref, be_ref):
    eps = 1e-10
    logits = lax.dot_general(wr_ref[...], cond_ref[...],
                             (((1,), (1,)), ((), ())),
                             preferred_element_type=jnp.float32) + br_ref[...]
    logits_ref[...] = logits
    g = -jnp.log(-jnp.log(gum_ref[...] + eps) + eps)
    z = (logits + g) / TAU
    m = jnp.max(z, axis=0, keepdims=True)
    ez = jnp.exp(z - m)
    gates_ref[...] = ez / jnp.sum(ez, axis=0, keepdims=True)

    is_max = (z == m).astype(jnp.float32)
    ir = lax.broadcasted_iota(jnp.int32, (E, E), 0)
    ic = lax.broadcasted_iota(jnp.int32, (E, E), 1)
    l_incl = (ic <= ir).astype(jnp.float32)
    l_strict = (ic < ir).astype(jnp.float32)
    prefix = jnp.dot(l_incl, is_max, preferred_element_type=jnp.float32)
    onehot = is_max * (prefix == 1.0).astype(jnp.float32)

    counts = jnp.sum(onehot, axis=1, keepdims=True)
    counts_ref[...] = counts / B

    c = onehot
    k = 1
    while k < B:
        sh = jnp.concatenate([jnp.zeros((E, k), jnp.float32), c[:, : B - k]],
                             axis=1)
        c = c + sh
        k *= 2
    excl = c - onehot
    rank = jnp.sum(excl * onehot, axis=0, keepdims=True)

    nb = jnp.floor((counts + (BLOCK - 1)) * (1.0 / BLOCK))
    starts = jnp.dot(l_strict, nb, preferred_element_type=jnp.float32)
    ends = jnp.dot(l_incl, nb, preferred_element_type=jnp.float32)
    base = jnp.sum(onehot * starts, axis=0, keepdims=True) * BLOCK
    dest_ref[...] = (base + rank).astype(jnp.int32)

    total = jnp.sum(nb, axis=0, keepdims=True)
    biota = lax.broadcasted_iota(jnp.int32, (E, BE_PAD), 1).astype(jnp.float32)
    obi = jnp.minimum(biota, jnp.maximum(total - 1.0, 0.0))
    ge = (obi >= ends).astype(jnp.float32)
    be = jnp.minimum(jnp.sum(ge, axis=0, keepdims=True), E - 1)
    be_ref[...] = jnp.concatenate(
        [be, obi[:1]], axis=0).astype(jnp.int32)


def _router(cond, gum_t, wr_t, br_c):
    return pl.pallas_call(
        _router_body,
        out_shape=(
            jax.ShapeDtypeStruct((E, B), jnp.float32),
            jax.ShapeDtypeStruct((E, B), jnp.float32),
            jax.ShapeDtypeStruct((E, 1), jnp.float32),
            jax.ShapeDtypeStruct((1, B), jnp.int32),
            jax.ShapeDtypeStruct((2, BE_PAD), jnp.int32),
        ),
    )(cond, gum_t, wr_t, br_c)


def _dispatch_body(noise_hbm, cond_hbm, dest_hbm, xs_hbm, idx_v, rows_v, sem):
    wid = lax.axis_index("s") * 2 + lax.axis_index("c")
    base = wid * TPW
    pltpu.sync_copy(dest_hbm.at[pl.ds(base, TPW)], idx_v)
    pltpu.sync_copy(noise_hbm.at[pl.ds(base, TPW)],
                    rows_v.at[:, pl.ds(0, D_NOISE)])
    pltpu.sync_copy(cond_hbm.at[pl.ds(base, TPW)],
                    rows_v.at[:, pl.ds(D_NOISE, D_COND)])
    pltpu.async_copy(rows_v, xs_hbm.at[idx_v], sem).wait()


def _dispatch(noise, cond, dest):
    mesh = plsc.VectorSubcoreMesh(core_axis_name="c", subcore_axis_name="s",
                                  num_cores=2, num_subcores=16)
    return pl.kernel(
        _dispatch_body,
        out_type=jax.ShapeDtypeStruct((NSLOT, D_IN), jnp.float32),
        mesh=mesh,
        scratch_types=[
            pltpu.VMEM((TPW,), jnp.int32),
            pltpu.VMEM((TPW, D_IN), jnp.float32),
            pltpu.SemaphoreType.DMA,
        ],
    )(noise, cond, dest)


def _combine_body(ys_hbm, dest_hbm, out_hbm, idx_v, rows_v, sem):
    wid = lax.axis_index("s") * 2 + lax.axis_index("c")
    base = wid * TPW
    pltpu.sync_copy(dest_hbm.at[pl.ds(base, TPW)], idx_v)
    pltpu.async_copy(ys_hbm.at[idx_v], rows_v, sem).wait()
    pltpu.sync_copy(rows_v, out_hbm.at[pl.ds(base, TPW)])


def _combine(ys, dest):
    mesh = plsc.VectorSubcoreMesh(core_axis_name="c", subcore_axis_name="s",
                                  num_cores=2, num_subcores=16)
    return pl.kernel(
        _combine_body,
        out_type=jax.ShapeDtypeStruct((B, D_PAD), jnp.float32),
        mesh=mesh,
        scratch_types=[
            pltpu.VMEM((TPW,), jnp.int32),
            pltpu.VMEM((TPW, D_PAD), jnp.float32),
            pltpu.SemaphoreType.DMA,
        ],
    )(ys, dest)


def _experts_body(be_ref, xs_ref, w1_ref, b1_ref, w2t_ref, b2_ref, out_ref):
    i = pl.program_id(0)

    @pl.when(be_ref[1, i] == i)
    def _():
        h = jnp.dot(xs_ref[...], w1_ref[0],
                    preferred_element_type=jnp.float32) + b1_ref[0]
        h = jnp.maximum(h, 0.0)
        y = lax.dot_general(h, w2t_ref[0], (((1,), (1,)), ((), ())),
                            preferred_element_type=jnp.float32) + b2_ref[0]
        out_ref[:, :D_OUT] = jnp.tanh(y)


def _experts(be, xs, W1, b1r, W2t, b2r):
    grid_spec = pltpu.PrefetchScalarGridSpec(
        num_scalar_prefetch=1,
        grid=(NBLK,),
        in_specs=[
            pl.BlockSpec((BLOCK, D_IN), lambda i, be: (be[1, i], 0)),
            pl.BlockSpec((1, D_IN, H), lambda i, be: (be[0, i], 0, 0)),
            pl.BlockSpec((1, 1, H), lambda i, be: (be[0, i], 0, 0)),
            pl.BlockSpec((1, D_OUT, H), lambda i, be: (be[0, i], 0, 0)),
            pl.BlockSpec((1, 1, D_OUT), lambda i, be: (be[0, i], 0, 0)),
        ],
        out_specs=pl.BlockSpec((BLOCK, D_PAD), lambda i, be: (be[1, i], 0)),
    )
    return pl.pallas_call(
        _experts_body,
        grid_spec=grid_spec,
        out_shape=jax.ShapeDtypeStruct((NSLOT, D_PAD), jnp.float32),
    )(be, xs, W1, b1r, W2t, b2r)


def kernel(cond, noise, gumbel_u, W_router, b_router, W1, b1, W2, b2):
    logits_t, gates_t, counts_c, dest_t, be = _router(
        cond, gumbel_u.T, W_router.T, b_router.reshape(E, 1))
    dest = dest_t.reshape(B)
    xs = _dispatch(noise, cond, dest)
    ys = _experts(be, xs, W1, b1.reshape(E, 1, H),
                  jnp.swapaxes(W2, 1, 2), b2.reshape(E, 1, D_OUT))
    outp = _combine(ys, dest)
    return (outp[:, :D_OUT], gates_t.T, logits_t.T, counts_c.reshape(E))
